# baseline (speedup 1.0000x reference)
import functools
import math

import jax
import jax.numpy as jnp
from jax import lax
from jax.experimental import pallas as pl
from jax.experimental.pallas import tpu as pltpu
from jax.experimental.pallas import tpu_sc as plsc

VOCAB = 100000
EMB = 128
SCALE = math.sqrt(float(EMB))

NC = 2
NS = 16
NW = NC * NS

CHUNK = 128
NBUF = 5


def _gather_kernel_body(n_slots, table_hbm, idx_hbm, out_hbm, idx_v, rows_v,
                        ga_sems, cp_sems):
    wid = lax.axis_index("s") * NC + lax.axis_index("c")
    idx_base = wid * n_slots
    out_base = wid * (n_slots * CHUNK)

    pltpu.sync_copy(idx_hbm.at[pl.ds(idx_base, n_slots)], idx_v)

    def issue_gather(g, b):
        pltpu.async_copy(table_hbm.at[idx_v.at[g]], rows_v.at[b], ga_sems[b])

    def wait_gather(g, b):
        pltpu.make_async_copy(
            table_hbm.at[idx_v.at[g]], rows_v.at[b], ga_sems[b]).wait()

    def scale_buf(b):
        @pl.loop(0, CHUNK, unroll=4)
        def _row(r):
            for j in range(EMB // 16):
                sl = pl.ds(j * 16, 16)
                rows_v[b, r, sl] = rows_v[b, r, sl] * SCALE

    def issue_copyout(h, b):
        pltpu.async_copy(
            rows_v.at[b], out_hbm.at[pl.ds(out_base + h * CHUNK, CHUNK)],
            cp_sems[b])

    def wait_copyout(h, b):
        pltpu.make_async_copy(
            rows_v.at[b], out_hbm.at[pl.ds(out_base + h * CHUNK, CHUNK)],
            cp_sems[b]).wait()

    for b in range(NBUF):
        issue_gather(b, b)
    wait_gather(0, 0)
    scale_buf(0)
    issue_copyout(0, 0)

    @pl.loop(1, n_slots // NBUF)
    def _grp(grp):
        for b in range(NBUF):
            g = grp * NBUF + b
            wait_copyout(g - NBUF, b)
            issue_gather(g, b)
            h = g - (NBUF - 1)
            bh = (b + 1) % NBUF
            wait_gather(h, bh)
            scale_buf(bh)
            issue_copyout(h, bh)

    for h in range(n_slots - (NBUF - 1), n_slots):
        bh = h % NBUF
        wait_gather(h, bh)
        scale_buf(bh)
        issue_copyout(h, bh)
    for h in range(n_slots - NBUF, n_slots):
        wait_copyout(h, h % NBUF)


def _sc_gather(table, idx_2d, n_slots):
    total_rows = NW * n_slots * CHUNK
    mesh = plsc.VectorSubcoreMesh(
        core_axis_name="c", subcore_axis_name="s", num_cores=NC,
        num_subcores=NS)
    kern = pl.kernel(
        functools.partial(_gather_kernel_body, n_slots),
        out_type=jax.ShapeDtypeStruct((total_rows, EMB), jnp.float32),
        mesh=mesh,
        scratch_types=[
            pltpu.VMEM((n_slots, CHUNK), jnp.int32),
            pltpu.VMEM((NBUF, CHUNK, EMB), jnp.float32),
            [pltpu.SemaphoreType.DMA] * NBUF,
            [pltpu.SemaphoreType.DMA] * NBUF,
        ],
    )
    return kern(table, idx_2d)


def kernel(tokens, emb_weight):
    b, l = tokens.shape
    total = b * l
    assert total % (NW * CHUNK) == 0
    n_slots = total // (NW * CHUNK)
    idx_2d = jnp.asarray(tokens, jnp.int32).reshape(NW * n_slots, CHUNK)
    table = jnp.asarray(emb_weight, jnp.float32)
    out = _sc_gather(table, idx_2d, n_slots)
    return out.reshape(b, l, EMB)
